# Initial kernel scaffold; baseline (speedup 1.0000x reference)
import functools

import jax
import jax.numpy as jnp
from jax import lax
from jax.experimental import pallas as pl
from jax.experimental.pallas import tpu as pltpu
from jax.experimental.pallas import tpu_sc as plsc

NODES = 50000
DIM = 64
EDGES = 800000

_NC = 2
_NS = 16
_NW = _NC * _NS
_EPW = EDGES // _NW
_C = 128
_NFULL = _EPW // _C
_REM = _EPW - _NFULL * _C


@functools.partial(
    pl.kernel,
    out_type=jax.ShapeDtypeStruct((EDGES,), jnp.float32),
    mesh=plsc.VectorSubcoreMesh(core_axis_name="c", subcore_axis_name="s"),
    scratch_types=[
        pltpu.VMEM((_EPW,), jnp.int32),
        pltpu.VMEM((_EPW,), jnp.int32),
        pltpu.VMEM((_C, DIM), jnp.float32),
        pltpu.VMEM((_C, DIM), jnp.float32),
        pltpu.VMEM((_EPW,), jnp.float32),
        pltpu.SemaphoreType.DMA,
        pltpu.SemaphoreType.DMA,
    ],
)
def _edge_dot(src_hbm, dst_hbm, w_hbm, out_hbm,
              idx_s, idx_d, rows_s, rows_d, out_v, sem_s, sem_d):
    wid = lax.axis_index("s") * _NC + lax.axis_index("c")
    base0 = wid * _EPW

    pltpu.sync_copy(src_hbm.at[pl.ds(base0, _EPW)], idx_s)
    pltpu.sync_copy(dst_hbm.at[pl.ds(base0, _EPW)], idx_d)

    def do_chunk(local_base, n):
        cs = pltpu.async_copy(
            w_hbm.at[idx_s.at[pl.ds(local_base, n)]],
            rows_s.at[pl.ds(0, n)], sem_s)
        cd = pltpu.async_copy(
            w_hbm.at[idx_d.at[pl.ds(local_base, n)]],
            rows_d.at[pl.ds(0, n)], sem_d)
        cs.wait()
        cd.wait()

        def edge_body(e, _):
            p = rows_s[e, pl.ds(0, 16)] * rows_d[e, pl.ds(0, 16)]
            p += rows_s[e, pl.ds(16, 16)] * rows_d[e, pl.ds(16, 16)]
            p += rows_s[e, pl.ds(32, 16)] * rows_d[e, pl.ds(32, 16)]
            p += rows_s[e, pl.ds(48, 16)] * rows_d[e, pl.ds(48, 16)]
            out_v[local_base + e] = jnp.sum(p)
            return _

        lax.fori_loop(0, n, edge_body, None)

    def chunk_loop(i, _):
        do_chunk(i * _C, _C)
        return _

    lax.fori_loop(0, _NFULL, chunk_loop, None)
    do_chunk(_NFULL * _C, _REM)

    pltpu.sync_copy(out_v, out_hbm.at[pl.ds(base0, _EPW)])


def kernel(edge_label_index, weight):
    src = edge_label_index[0]
    dst = edge_label_index[1]
    return _edge_dot(src, dst, weight)


# speedup vs baseline: 13.2807x; 13.2807x over previous
import functools

import jax
import jax.numpy as jnp
from jax import lax
from jax.experimental import pallas as pl
from jax.experimental.pallas import tpu as pltpu
from jax.experimental.pallas import tpu_sc as plsc

NODES = 50000
DIM = 64
EDGES = 800000

_NC = 2
_NS = 16
_NW = _NC * _NS
_EPW = EDGES // _NW
_C = 128
_NFULL = _EPW // _C
_REM = _EPW - _NFULL * _C
_NPAIR = (_NFULL - 1) // 2


@functools.partial(
    pl.kernel,
    out_type=jax.ShapeDtypeStruct((EDGES,), jnp.float32),
    mesh=plsc.VectorSubcoreMesh(core_axis_name="c", subcore_axis_name="s"),
    compiler_params=pltpu.CompilerParams(use_tc_tiling_on_sc=False),
    scratch_types=[
        pltpu.VMEM((_EPW,), jnp.int32),
        pltpu.VMEM((_EPW,), jnp.int32),
        pltpu.VMEM((_C, DIM), jnp.float32),
        pltpu.VMEM((_C, DIM), jnp.float32),
        pltpu.VMEM((_C, DIM), jnp.float32),
        pltpu.VMEM((_C, DIM), jnp.float32),
        pltpu.VMEM((_EPW,), jnp.float32),
        pltpu.SemaphoreType.DMA,
        pltpu.SemaphoreType.DMA,
        pltpu.SemaphoreType.DMA,
        pltpu.SemaphoreType.DMA,
    ],
)
def _edge_dot(src_hbm, dst_hbm, w_hbm, out_hbm,
              idx_s, idx_d, rs0, rd0, rs1, rd1, out_v,
              ss0, sd0, ss1, sd1):
    wid = lax.axis_index("s") * _NC + lax.axis_index("c")
    base0 = wid * _EPW

    pltpu.sync_copy(src_hbm.at[pl.ds(base0, _EPW)], idx_s)
    pltpu.sync_copy(dst_hbm.at[pl.ds(base0, _EPW)], idx_d)

    lane = lax.iota(jnp.int32, 16)
    rot_idx = [((lane + (1 << k)) & 15).reshape(16, 1) for k in range(4)]
    _dnums = lax.GatherDimensionNumbers(
        offset_dims=(), collapsed_slice_dims=(0,), start_index_map=(0,))

    def hsum(p):
        for k in range(4):
            p = p + lax.gather(
                p, rot_idx[k], _dnums, (1,),
                mode=lax.GatherScatterMode.PROMISE_IN_BOUNDS)
        return p

    def start(lb, n, bs, bd, ss, sd):
        pltpu.async_copy(
            w_hbm.at[idx_s.at[pl.ds(lb, n)]], bs.at[pl.ds(0, n)], ss)
        pltpu.async_copy(
            w_hbm.at[idx_d.at[pl.ds(lb, n)]], bd.at[pl.ds(0, n)], sd)

    def wait(n, bs, bd, ss, sd):
        pltpu.make_async_copy(
            w_hbm.at[idx_s.at[pl.ds(0, n)]], bs.at[pl.ds(0, n)], ss).wait()
        pltpu.make_async_copy(
            w_hbm.at[idx_d.at[pl.ds(0, n)]], bd.at[pl.ds(0, n)], sd).wait()

    def compute(local_base, bs, bd, ngroups, tail):
        def edge_total(e):
            p = bs[e, pl.ds(0, 16)] * bd[e, pl.ds(0, 16)]
            p += bs[e, pl.ds(16, 16)] * bd[e, pl.ds(16, 16)]
            p += bs[e, pl.ds(32, 16)] * bd[e, pl.ds(32, 16)]
            p += bs[e, pl.ds(48, 16)] * bd[e, pl.ds(48, 16)]
            return hsum(p)

        def do_group(start_e):
            res = jnp.zeros((16,), jnp.float32)
            for l in range(16):
                res = jnp.where(lane == l, edge_total(start_e + l), res)
            out_v[pl.ds(local_base + start_e, 16)] = res

        lax.fori_loop(0, ngroups, lambda g, _: (do_group(g * 16), _)[1], None)
        if tail:
            do_group(ngroups * 16 + tail - 16)

    start(0, _C, rs0, rd0, ss0, sd0)

    def pair_body(k, _):
        c0 = (2 * k) * _C
        start(c0 + _C, _C, rs1, rd1, ss1, sd1)
        wait(_C, rs0, rd0, ss0, sd0)
        compute(c0, rs0, rd0, _C // 16, 0)
        start(c0 + 2 * _C, _C, rs0, rd0, ss0, sd0)
        wait(_C, rs1, rd1, ss1, sd1)
        compute(c0 + _C, rs1, rd1, _C // 16, 0)
        return _

    lax.fori_loop(0, _NPAIR, pair_body, None)

    last_full = (_NFULL - 1) * _C
    start(_NFULL * _C, _REM, rs1, rd1, ss1, sd1)
    wait(_C, rs0, rd0, ss0, sd0)
    compute(last_full, rs0, rd0, _C // 16, 0)
    wait(_REM, rs1, rd1, ss1, sd1)
    compute(_NFULL * _C, rs1, rd1, _REM // 16, _REM % 16)

    pltpu.sync_copy(out_v, out_hbm.at[pl.ds(base0, _EPW)])


def kernel(edge_label_index, weight):
    src = edge_label_index[0]
    dst = edge_label_index[1]
    return _edge_dot(src, dst, weight)
